# Initial kernel scaffold; baseline (speedup 1.0000x reference)
import jax
import jax.numpy as jnp
from jax.experimental import pallas as pl


def kernel(x, W1, b1, W2, b2):
    raise NotImplementedError("write your pallas kernel here")


# speedup vs baseline: 7.8017x; 7.8017x over previous
import functools

import jax
import jax.numpy as jnp
from jax import lax
from jax.experimental import pallas as pl
from jax.experimental.pallas import tpu as pltpu

IN_DIM = 768
HIDDEN_DIM = 16
NUM_EXP = 64
TOPK = 8
EPS = 1e-12

BM = 2048


def _gate_block(x_ref, w1_ref, b1_ref, w2_ref, b2_ref, o_ref):
    x = x_ref[...]
    h = jnp.maximum(
        jnp.dot(x, w1_ref[...], preferred_element_type=jnp.float32) + b1_ref[...],
        0.0,
    )
    logits = jnp.dot(h, w2_ref[...], preferred_element_type=jnp.float32) + b2_ref[...]

    col = lax.broadcasted_iota(jnp.int32, logits.shape, 1)
    neg = jnp.float32(-3.4e38)
    cur = logits
    keep = jnp.zeros(logits.shape, dtype=jnp.bool_)
    row_max = jnp.max(logits, axis=-1, keepdims=True)
    for _ in range(TOPK):
        m = jnp.max(cur, axis=-1, keepdims=True)
        at_m = cur == m
        first = jnp.min(jnp.where(at_m, col, NUM_EXP), axis=-1, keepdims=True)
        sel = col == first
        keep = jnp.logical_or(keep, sel)
        cur = jnp.where(sel, neg, cur)

    e = jnp.exp(logits - row_max)
    z_all = jnp.sum(e, axis=-1, keepdims=True)
    ek = jnp.where(keep, e, 0.0)
    s = jnp.sum(ek, axis=-1, keepdims=True)
    o_ref[...] = ek / (s + EPS * z_all)


@jax.jit
def kernel(x, W1, b1, W2, b2):
    b = x.shape[0]
    grid = (b // BM,)
    return pl.pallas_call(
        _gate_block,
        grid=grid,
        in_specs=[
            pl.BlockSpec((BM, IN_DIM), lambda i: (i, 0)),
            pl.BlockSpec((IN_DIM, HIDDEN_DIM), lambda i: (0, 0)),
            pl.BlockSpec((1, HIDDEN_DIM), lambda i: (0, 0)),
            pl.BlockSpec((HIDDEN_DIM, NUM_EXP), lambda i: (0, 0)),
            pl.BlockSpec((1, NUM_EXP), lambda i: (0, 0)),
        ],
        out_specs=pl.BlockSpec((BM, NUM_EXP), lambda i: (i, 0)),
        out_shape=jax.ShapeDtypeStruct((b, NUM_EXP), jnp.float32),
        compiler_params=pltpu.CompilerParams(
            dimension_semantics=("arbitrary",),
        ),
    )(x, W1, b1.reshape(1, HIDDEN_DIM), W2, b2.reshape(1, NUM_EXP))


# speedup vs baseline: 9.8386x; 1.2611x over previous
import functools

import jax
import jax.numpy as jnp
from jax import lax
from jax.experimental import pallas as pl
from jax.experimental.pallas import tpu as pltpu

IN_DIM = 768
HIDDEN_DIM = 16
NUM_EXP = 64
TOPK = 8
EPS = 1e-12

BM = 2048


def _gate_block(x_ref, w1_ref, b1_ref, w2_ref, b2_ref, o_ref):
    x = x_ref[...]
    h = jnp.maximum(
        jnp.dot(x, w1_ref[...], preferred_element_type=jnp.float32) + b1_ref[...],
        0.0,
    )
    logits = jnp.dot(h, w2_ref[...], preferred_element_type=jnp.float32) + b2_ref[...]

    bits = lax.bitcast_convert_type(logits, jnp.int32)
    key = bits ^ lax.shift_right_logical(
        lax.shift_right_arithmetic(bits, 31), 1
    )
    int_min = jnp.int32(-2147483648)
    keep = jnp.zeros(logits.shape, dtype=jnp.bool_)
    m0 = jnp.max(key, axis=-1, keepdims=True)
    cur = key
    m = m0
    for i in range(TOPK):
        sel = cur == m
        keep = jnp.logical_or(keep, sel)
        if i < TOPK - 1:
            cur = jnp.where(sel, int_min, cur)
            m = jnp.max(cur, axis=-1, keepdims=True)

    row_max = lax.bitcast_convert_type(
        m0 ^ lax.shift_right_logical(lax.shift_right_arithmetic(m0, 31), 1),
        jnp.float32,
    )
    ek = jnp.where(keep, jnp.exp(logits - row_max), 0.0)
    s = jnp.sum(ek, axis=-1, keepdims=True)
    o_ref[...] = ek / s


@jax.jit
def kernel(x, W1, b1, W2, b2):
    b = x.shape[0]
    grid = (b // BM,)
    return pl.pallas_call(
        _gate_block,
        grid=grid,
        in_specs=[
            pl.BlockSpec((BM, IN_DIM), lambda i: (i, 0)),
            pl.BlockSpec((IN_DIM, HIDDEN_DIM), lambda i: (0, 0)),
            pl.BlockSpec((1, HIDDEN_DIM), lambda i: (0, 0)),
            pl.BlockSpec((HIDDEN_DIM, NUM_EXP), lambda i: (0, 0)),
            pl.BlockSpec((1, NUM_EXP), lambda i: (0, 0)),
        ],
        out_specs=pl.BlockSpec((BM, NUM_EXP), lambda i: (i, 0)),
        out_shape=jax.ShapeDtypeStruct((b, NUM_EXP), jnp.float32),
        compiler_params=pltpu.CompilerParams(
            dimension_semantics=("arbitrary",),
        ),
    )(x, W1, b1.reshape(1, HIDDEN_DIM), W2, b2.reshape(1, NUM_EXP))


# speedup vs baseline: 15.3277x; 1.5579x over previous
import functools

import jax
import jax.numpy as jnp
from jax import lax
from jax.experimental import pallas as pl
from jax.experimental.pallas import tpu as pltpu

IN_DIM = 768
HIDDEN_DIM = 16
NUM_EXP = 64
TOPK = 8
EPS = 1e-12

BM = 2048


def _gate_block(x_ref, w1_ref, b1_ref, w2_ref, b2_ref, o_ref):
    x = x_ref[...]
    h = jnp.maximum(
        jnp.dot(x, w1_ref[...], preferred_element_type=jnp.float32) + b1_ref[...],
        0.0,
    )
    logits = jnp.dot(h, w2_ref[...], preferred_element_type=jnp.float32) + b2_ref[...]

    neg = jnp.float32(-3.4e38)
    keep = jnp.zeros(logits.shape, dtype=jnp.bool_)
    row_max = jnp.max(logits, axis=-1, keepdims=True)
    cur = logits
    m = row_max
    for i in range(TOPK):
        sel = cur == m
        keep = jnp.logical_or(keep, sel)
        if i < TOPK - 1:
            cur = jnp.where(sel, neg, cur)
            m = jnp.max(cur, axis=-1, keepdims=True)

    ek = jnp.where(keep, jnp.exp(logits - row_max), 0.0)
    s = jnp.sum(ek, axis=-1, keepdims=True)
    o_ref[...] = ek / s


@jax.jit
def kernel(x, W1, b1, W2, b2):
    b = x.shape[0]
    grid = (b // BM,)
    return pl.pallas_call(
        _gate_block,
        grid=grid,
        in_specs=[
            pl.BlockSpec((BM, IN_DIM), lambda i: (i, 0)),
            pl.BlockSpec((IN_DIM, HIDDEN_DIM), lambda i: (0, 0)),
            pl.BlockSpec((1, HIDDEN_DIM), lambda i: (0, 0)),
            pl.BlockSpec((HIDDEN_DIM, NUM_EXP), lambda i: (0, 0)),
            pl.BlockSpec((1, NUM_EXP), lambda i: (0, 0)),
        ],
        out_specs=pl.BlockSpec((BM, NUM_EXP), lambda i: (i, 0)),
        out_shape=jax.ShapeDtypeStruct((b, NUM_EXP), jnp.float32),
        compiler_params=pltpu.CompilerParams(
            dimension_semantics=("arbitrary",),
        ),
    )(x, W1, b1.reshape(1, HIDDEN_DIM), W2, b2.reshape(1, NUM_EXP))


# speedup vs baseline: 16.3186x; 1.0646x over previous
import functools

import jax
import jax.numpy as jnp
from jax import lax
from jax.experimental import pallas as pl
from jax.experimental.pallas import tpu as pltpu

IN_DIM = 768
HIDDEN_DIM = 16
NUM_EXP = 64
TOPK = 8
EPS = 1e-12

BM = 2048


def _gate_block(x_ref, w1_ref, b1_ref, w2_ref, b2_ref, o_ref):
    x = x_ref[...]
    h = jnp.maximum(
        jnp.dot(x, w1_ref[...], preferred_element_type=jnp.float32) + b1_ref[...],
        0.0,
    )
    logits = jnp.dot(h, w2_ref[...], preferred_element_type=jnp.float32) + b2_ref[...]

    neg = jnp.float32(-3.4e38)
    row_max = jnp.max(logits, axis=-1, keepdims=True)
    m = row_max
    for _ in range(TOPK - 1):
        cur = jnp.where(logits >= m, neg, logits)
        m = jnp.max(cur, axis=-1, keepdims=True)

    ek = jnp.where(logits >= m, jnp.exp(logits - row_max), 0.0)
    s = jnp.sum(ek, axis=-1, keepdims=True)
    o_ref[...] = ek / s


@jax.jit
def kernel(x, W1, b1, W2, b2):
    b = x.shape[0]
    grid = (b // BM,)
    return pl.pallas_call(
        _gate_block,
        grid=grid,
        in_specs=[
            pl.BlockSpec((BM, IN_DIM), lambda i: (i, 0)),
            pl.BlockSpec((IN_DIM, HIDDEN_DIM), lambda i: (0, 0)),
            pl.BlockSpec((1, HIDDEN_DIM), lambda i: (0, 0)),
            pl.BlockSpec((HIDDEN_DIM, NUM_EXP), lambda i: (0, 0)),
            pl.BlockSpec((1, NUM_EXP), lambda i: (0, 0)),
        ],
        out_specs=pl.BlockSpec((BM, NUM_EXP), lambda i: (i, 0)),
        out_shape=jax.ShapeDtypeStruct((b, NUM_EXP), jnp.float32),
        compiler_params=pltpu.CompilerParams(
            dimension_semantics=("arbitrary",),
        ),
    )(x, W1, b1.reshape(1, HIDDEN_DIM), W2, b2.reshape(1, NUM_EXP))


# speedup vs baseline: 16.4529x; 1.0082x over previous
import functools

import jax
import jax.numpy as jnp
from jax import lax
from jax.experimental import pallas as pl
from jax.experimental.pallas import tpu as pltpu

IN_DIM = 768
HIDDEN_DIM = 16
NUM_EXP = 64
TOPK = 8
EPS = 1e-12

BM = 4096


def _gate_block(x_ref, w1_ref, b1_ref, w2_ref, b2_ref, o_ref):
    x = x_ref[...]
    h = jnp.maximum(
        jnp.dot(x, w1_ref[...], preferred_element_type=jnp.float32) + b1_ref[...],
        0.0,
    )
    logits = jnp.dot(h, w2_ref[...], preferred_element_type=jnp.float32) + b2_ref[...]

    neg = jnp.float32(-3.4e38)
    row_max = jnp.max(logits, axis=-1, keepdims=True)
    m = row_max
    for _ in range(TOPK - 1):
        cur = jnp.where(logits >= m, neg, logits)
        m = jnp.max(cur, axis=-1, keepdims=True)

    ek = jnp.where(logits >= m, jnp.exp(logits - row_max), 0.0)
    s = jnp.sum(ek, axis=-1, keepdims=True)
    o_ref[...] = ek / s


@jax.jit
def kernel(x, W1, b1, W2, b2):
    b = x.shape[0]
    grid = (b // BM,)
    return pl.pallas_call(
        _gate_block,
        grid=grid,
        in_specs=[
            pl.BlockSpec((BM, IN_DIM), lambda i: (i, 0)),
            pl.BlockSpec((IN_DIM, HIDDEN_DIM), lambda i: (0, 0)),
            pl.BlockSpec((1, HIDDEN_DIM), lambda i: (0, 0)),
            pl.BlockSpec((HIDDEN_DIM, NUM_EXP), lambda i: (0, 0)),
            pl.BlockSpec((1, NUM_EXP), lambda i: (0, 0)),
        ],
        out_specs=pl.BlockSpec((BM, NUM_EXP), lambda i: (i, 0)),
        out_shape=jax.ShapeDtypeStruct((b, NUM_EXP), jnp.float32),
        compiler_params=pltpu.CompilerParams(
            dimension_semantics=("arbitrary",),
        ),
    )(x, W1, b1.reshape(1, HIDDEN_DIM), W2, b2.reshape(1, NUM_EXP))
